# baseline (speedup 1.0000x reference)
import functools
import numpy as np
import jax
import jax.numpy as jnp
from jax import lax
from jax.experimental import pallas as pl
from jax.experimental.pallas import tpu as pltpu
from jax.experimental.pallas import tpu_sc as plsc

RES = 23
KERNELS = [1, 3, 5]
WBASE = 16
NG = RES + 1
NDOF = 2 * NG * NG
NEL = RES * RES
STRIP = 48 * NDOF
DUMP = STRIP
ACC_N = STRIP + 128


def _build_sc_idx(k):
    pad = (k - 1) // 2
    s = k + 1
    L = 2 * s * s
    sh = (NG, s, RES, s, 2, s, s, 2)
    R1, W, COL, CA, PA, RB, CB, PB = np.ogrid[0:NG, 0:s, 0:RES, 0:s, 0:2, 0:s, 0:s, 0:2]
    row = R1 + pad - k + W
    c1 = COL + CA - pad
    r2 = row + RB - pad
    c2 = COL + CB - pad
    ok = ((row >= 0) & (row <= RES - 1) & (c1 >= 0) & (c1 < NG)
          & (r2 >= 0) & (r2 < NG) & (c2 >= 0) & (c2 < NG))
    val = (2 * c1 + PA) * NDOF + 48 * r2 + 2 * c2 + PB
    flat = ((COL * 2 * s + CA * 2 + PA) * L + (RB * s + CB) * 2 + PB) % 16
    idx = np.where(ok, val, DUMP + flat)
    idx = np.broadcast_to(idx, sh)
    return np.ascontiguousarray(idx.reshape(NG, s, RES, 2 * s * L).astype(np.int32))


_SC_IDX = {k: _build_sc_idx(k) for k in KERNELS}


def _tril_cols(out_size):
    tri = np.tril_indices(out_size)
    return tri[0] * out_size + tri[1]


def _tower_body(Lk, p_ref, w1_ref, b1_ref, w2_ref, b2_ref, w3_ref, b3_ref,
                wm_ref, out_ref):
    P = p_ref[0]
    H1 = jnp.maximum(jnp.dot(P, w1_ref[...], preferred_element_type=jnp.float32)
                     + b1_ref[...], 0.0)
    H2 = jnp.maximum(jnp.dot(H1, w2_ref[...], preferred_element_type=jnp.float32)
                     + b2_ref[...], 0.0)
    H3 = jnp.maximum(jnp.dot(H2, w3_ref[...], preferred_element_type=jnp.float32)
                     + b3_ref[...], 0.0)
    Bm = jnp.dot(H3, wm_ref[...], preferred_element_type=jnp.float32)
    ii = lax.broadcasted_iota(jnp.int32, (1, Lk * Lk), 1)
    dmask = (ii // Lk) == (ii % Lk)
    Bm = jnp.where(dmask, jnp.maximum(Bm, 0.0), Bm)
    Bm3 = Bm.reshape(NEL, Lk, Lk)
    P = {72: 3, 32: 8, 8: 32}[Lk]
    i = 0
    while i < NEL:
        p = min(P, NEL - i)
        A = Bm3[i:i + p].reshape(p * Lk, Lk)
        G = lax.dot_general(A, A, (((1,), (1,)), ((), ())),
                            preferred_element_type=jnp.float32)
        for u in range(p):
            out_ref[0, i + u] = G[u * Lk:(u + 1) * Lk, u * Lk:(u + 1) * Lk]
        i += p


def _tower(k, patches, w1, b1, w2, b2, w3, b3, wm2):
    B = patches.shape[0]
    Ck = patches.shape[2]
    Lk = 2 * (k + 1) ** 2
    wi = w1.shape[1]
    full = lambda shp: pl.BlockSpec(shp, lambda b: (0,) * len(shp))
    return pl.pallas_call(
        functools.partial(_tower_body, Lk),
        grid=(B,),
        in_specs=[
            pl.BlockSpec((1, NEL, Ck), lambda b: (b, 0, 0)),
            full((Ck, wi)), full((1, wi)),
            full((wi, 2 * wi)), full((1, 2 * wi)),
            full((2 * wi, 2 * wi)), full((1, 2 * wi)),
            full((2 * wi, Lk * Lk)),
        ],
        out_specs=pl.BlockSpec((1, NEL, Lk, Lk), lambda b: (b, 0, 0, 0)),
        out_shape=jax.ShapeDtypeStruct((B, NEL, Lk, Lk), jnp.float32),
        compiler_params=pltpu.CompilerParams(
            vmem_limit_bytes=100 * 1024 * 1024),
    )(patches, w1, b1, w2, b2, w3, b3, wm2)


def _sc_assemble(bm5, bm3, bm1):
    mesh = plsc.VectorSubcoreMesh(core_axis_name="c", subcore_axis_name="s")
    B = bm5.shape[0]

    @functools.partial(
        pl.kernel,
        out_type=jax.ShapeDtypeStruct((B, NG, STRIP), jnp.float32),
        mesh=mesh,
        compiler_params=pltpu.CompilerParams(needs_layout_passes=False),
        scratch_types=[
            pltpu.VMEM((ACC_N,), jnp.float32),
            pltpu.VMEM((16, 864), jnp.float32), pltpu.VMEM((16, 864), jnp.int32),
            pltpu.VMEM((16, 864), jnp.float32), pltpu.VMEM((16, 864), jnp.int32),
            pltpu.VMEM((RES, 32), jnp.float32), pltpu.VMEM((RES, 32), jnp.int32),
            pltpu.SemaphoreType.DMA, pltpu.SemaphoreType.DMA,
            pltpu.SemaphoreType.DMA, pltpu.SemaphoreType.DMA,
        ],
    )
    def asm(bm5_h, bm3_h, bm1_h, i5_h, i3_h, i1_h, out_h,
            acc, va, xa, vb, xb, v1, x1, sva, sxa, svb, sxb):
        wid = lax.axis_index("s") * 2 + lax.axis_index("c")
        zero = jnp.zeros((16,), jnp.float32)
        vbufs = (va, vb)
        xbufs = (xa, xb)
        svs = (sva, svb)
        sxs = (sxa, sxb)

        sched = []
        for kk, npv, unroll in ((5, 54, 6), (3, 16, 8)):
            for w in range(kk + 1):
                for c0, nc in ((0, 16), (16, 7)):
                    sched.append((kk, w, c0, nc, npv, unroll))

        def task_body(t, _):
            task = wid * 12 + t
            b = task // NG
            r1 = task % NG

            def zbody(i, _):
                for u in range(8):
                    acc[pl.ds(i * 128 + u * 16, 16)] = zero
                return 0
            lax.fori_loop(0, ACC_N // 128, zbody, 0)

            def issue(j, slot):
                kk, w, c0, nc, npv, unroll = sched[j]
                bm_h, i_h = (bm5_h, i5_h) if kk == 5 else (bm3_h, i3_h)
                wdt = 16 * npv
                pad = (kk - 1) // 2
                row = jnp.clip(r1 + pad - kk + w, 0, RES - 1)
                ra = kk - w
                cv = pltpu.async_copy(
                    bm_h.at[b, row, pl.ds(c0, nc), ra, :],
                    vbufs[slot].at[pl.ds(0, nc), pl.ds(0, wdt)], svs[slot])
                cx = pltpu.async_copy(
                    i_h.at[r1, w, pl.ds(c0, nc)],
                    xbufs[slot].at[pl.ds(0, nc), pl.ds(0, wdt)], sxs[slot])
                return cv, cx

            pend = issue(0, 0)
            for j, (kk, w, c0, nc, npv, unroll) in enumerate(sched):
                slot = j % 2
                cv, cx = pend
                cv.wait()
                cx.wait()
                if j + 1 < len(sched):
                    pend = issue(j + 1, 1 - slot)
                vbuf = vbufs[slot]
                xbuf = xbufs[slot]

                def cbody(col, _, vbuf=vbuf, xbuf=xbuf, npv=npv, unroll=unroll):
                    def jbody(jj, _):
                        for u in range(unroll):
                            off = jj * unroll * 16 + u * 16
                            iv = xbuf[col, pl.ds(off, 16)]
                            vv = vbuf[col, pl.ds(off, 16)]
                            plsc.addupdate_scatter(acc, [iv], vv)
                        return 0
                    lax.fori_loop(0, npv // unroll, jbody, 0)
                    return 0
                lax.fori_loop(0, nc, cbody, 0)

            for w in range(2):
                row = jnp.clip(r1 - 1 + w, 0, RES - 1)
                ra = 1 - w
                pltpu.sync_copy(bm1_h.at[b, row, :, ra, :], v1)
                pltpu.sync_copy(i1_h.at[r1, w], x1)

                def c1body(col, _):
                    for u in range(2):
                        iv = x1[col, pl.ds(u * 16, 16)]
                        vv = v1[col, pl.ds(u * 16, 16)]
                        plsc.addupdate_scatter(acc, [iv], vv)
                    return 0
                lax.fori_loop(0, RES, c1body, 0)

            pltpu.sync_copy(acc.at[pl.ds(0, STRIP)], out_h.at[b, r1])
            return 0
        lax.fori_loop(0, 12, task_body, 0)

    i5 = jnp.asarray(_SC_IDX[5])
    i3 = jnp.asarray(_SC_IDX[3])
    i1 = jnp.asarray(_SC_IDX[1])
    return asm(bm5, bm3, bm1, i5, i3, i1)


ROW_BLK = 128


def _mask_body(k_ref, zm_ref, out_ref):
    v = k_ref[...]
    out_ref[...] = jnp.where(zm_ref[...], jnp.zeros((), v.dtype), v)


def _mask(K, zero_map):
    B = K.shape[0]
    spec = pl.BlockSpec((1, ROW_BLK, NDOF), lambda b, r: (b, r, 0))
    return pl.pallas_call(
        _mask_body,
        grid=(B, NDOF // ROW_BLK),
        in_specs=[spec, spec],
        out_specs=spec,
        out_shape=jax.ShapeDtypeStruct((B, NDOF, NDOF), K.dtype),
    )(K, zero_map)


def _im2col(x, k):
    pad = (k - 1) // 2
    B = x.shape[0]
    xp = jnp.pad(x, ((0, 0), (0, 0), (pad, pad), (pad, pad)))
    cols = [xp[:, c, dy:dy + RES, dx:dx + RES]
            for c in range(2) for dy in range(k) for dx in range(k)]
    return jnp.stack(cols, axis=-1).reshape(B, NEL, 2 * k * k)


def kernel(x, zero_map, DBC, f,
           c11_w_1, c11_b_1, c12_w_1, c12_b_1, c13_w_1, c13_b_1, convt_w_1,
           c11_w_3, c11_b_3, c12_w_3, c12_b_3, c13_w_3, c13_b_3, convt_w_3,
           c11_w_5, c11_b_5, c12_w_5, c12_b_5, c13_w_5, c13_b_5, convt_w_5):
    params = {
        1: (c11_w_1, c11_b_1, c12_w_1, c12_b_1, c13_w_1, c13_b_1, convt_w_1),
        3: (c11_w_3, c11_b_3, c12_w_3, c12_b_3, c13_w_3, c13_b_3, convt_w_3),
        5: (c11_w_5, c11_b_5, c12_w_5, c12_b_5, c13_w_5, c13_b_5, convt_w_5),
    }
    bms = {}
    for k in KERNELS:
        w1, b1, w2, b2, w3, b3, cw = params[k]
        wi = w1.shape[0]
        out_size = 2 * (k + 1) ** 2
        W1 = w1.reshape(wi, 2 * k * k).T
        W2 = w2[:, :, 0, 0].T
        W3 = w3[:, :, 0, 0].T
        Wm = cw[:, 0, 0, :]
        Wm2 = jnp.zeros((2 * wi, out_size * out_size), Wm.dtype)
        Wm2 = Wm2.at[:, _tril_cols(out_size)].set(Wm)
        patches = _im2col(x, k)
        bm = _tower(k, patches, W1, b1.reshape(1, -1), W2, b2.reshape(1, -1),
                    W3, b3.reshape(1, -1), Wm2)
        s = k + 1
        L = 2 * s * s
        bms[k] = bm.reshape(x.shape[0], RES, RES, s, 2 * s * L)
    raw = _sc_assemble(bms[5], bms[3], bms[1])
    K = raw.reshape(x.shape[0], NDOF, NDOF)
    return _mask(K, zero_map)


# speedup vs baseline: 1.0044x; 1.0044x over previous
import functools
import numpy as np
import jax
import jax.numpy as jnp
from jax import lax
from jax.experimental import pallas as pl
from jax.experimental.pallas import tpu as pltpu
from jax.experimental.pallas import tpu_sc as plsc

RES = 23
KERNELS = [1, 3, 5]
WBASE = 16
NG = RES + 1
NDOF = 2 * NG * NG
NEL = RES * RES
STRIP = 48 * NDOF
DUMP = STRIP
ACC_N = STRIP + 128


def _build_sc_idx(k):
    pad = (k - 1) // 2
    s = k + 1
    L = 2 * s * s
    sh = (NG, s, RES, s, 2, s, s, 2)
    R1, W, COL, CA, PA, RB, CB, PB = np.ogrid[0:NG, 0:s, 0:RES, 0:s, 0:2, 0:s, 0:s, 0:2]
    row = R1 + pad - k + W
    c1 = COL + CA - pad
    r2 = row + RB - pad
    c2 = COL + CB - pad
    ok = ((row >= 0) & (row <= RES - 1) & (c1 >= 0) & (c1 < NG)
          & (r2 >= 0) & (r2 < NG) & (c2 >= 0) & (c2 < NG))
    val = (2 * c1 + PA) * NDOF + 48 * r2 + 2 * c2 + PB
    flat = ((COL * 2 * s + CA * 2 + PA) * L + (RB * s + CB) * 2 + PB) % 16
    idx = np.where(ok, val, DUMP + flat)
    idx = np.broadcast_to(idx, sh)
    return np.ascontiguousarray(idx.reshape(NG, s, RES, 2 * s * L).astype(np.int32))


_SC_IDX = {k: _build_sc_idx(k) for k in KERNELS}


def _tril_cols(out_size):
    tri = np.tril_indices(out_size)
    return tri[0] * out_size + tri[1]


NP = 136


def _tower_body(Lk, p_ref, w1_ref, b1_ref, w2_ref, b2_ref, w3_ref, b3_ref,
                wm_ref, out_ref):
    P = p_ref[0]
    H1 = jnp.maximum(jnp.dot(P, w1_ref[...], preferred_element_type=jnp.float32)
                     + b1_ref[...], 0.0)
    H2 = jnp.maximum(jnp.dot(H1, w2_ref[...], preferred_element_type=jnp.float32)
                     + b2_ref[...], 0.0)
    H3 = jnp.maximum(jnp.dot(H2, w3_ref[...], preferred_element_type=jnp.float32)
                     + b3_ref[...], 0.0)
    Bm = jnp.dot(H3, wm_ref[...], preferred_element_type=jnp.float32)
    ii = lax.broadcasted_iota(jnp.int32, (1, Lk * Lk), 1)
    dmask = (ii // Lk) == (ii % Lk)
    Bm = jnp.where(dmask, jnp.maximum(Bm, 0.0), Bm)
    Bm3 = Bm.reshape(NP, Lk, Lk)
    bm = lax.dot_general(Bm3, Bm3, (((2,), (2,)), ((0,), (0,))),
                         preferred_element_type=jnp.float32)
    out_ref[0] = bm


def _tower(k, patches, w1, b1, w2, b2, w3, b3, wm2):
    B = patches.shape[0]
    Ck = patches.shape[2]
    Lk = 2 * (k + 1) ** 2
    wi = w1.shape[1]
    full = lambda shp: pl.BlockSpec(shp, lambda b, c: (0,) * len(shp))
    return pl.pallas_call(
        functools.partial(_tower_body, Lk),
        grid=(B, pl.cdiv(NEL, NP)),
        in_specs=[
            pl.BlockSpec((1, NP, Ck), lambda b, c: (b, c, 0)),
            full((Ck, wi)), full((1, wi)),
            full((wi, 2 * wi)), full((1, 2 * wi)),
            full((2 * wi, 2 * wi)), full((1, 2 * wi)),
            full((2 * wi, Lk * Lk)),
        ],
        out_specs=pl.BlockSpec((1, NP, Lk, Lk), lambda b, c: (b, c, 0, 0)),
        out_shape=jax.ShapeDtypeStruct((B, NEL, Lk, Lk), jnp.float32),
        compiler_params=pltpu.CompilerParams(
            vmem_limit_bytes=100 * 1024 * 1024),
    )(patches, w1, b1, w2, b2, w3, b3, wm2)


def _sc_assemble(bm5, bm3, bm1):
    mesh = plsc.VectorSubcoreMesh(core_axis_name="c", subcore_axis_name="s")
    B = bm5.shape[0]

    @functools.partial(
        pl.kernel,
        out_type=jax.ShapeDtypeStruct((B, NG, STRIP), jnp.float32),
        mesh=mesh,
        compiler_params=pltpu.CompilerParams(needs_layout_passes=False),
        scratch_types=[
            pltpu.VMEM((ACC_N,), jnp.float32),
            pltpu.VMEM((16, 864), jnp.float32), pltpu.VMEM((16, 864), jnp.int32),
            pltpu.VMEM((16, 864), jnp.float32), pltpu.VMEM((16, 864), jnp.int32),
            pltpu.VMEM((RES, 32), jnp.float32), pltpu.VMEM((RES, 32), jnp.int32),
            pltpu.SemaphoreType.DMA, pltpu.SemaphoreType.DMA,
            pltpu.SemaphoreType.DMA, pltpu.SemaphoreType.DMA,
        ],
    )
    def asm(bm5_h, bm3_h, bm1_h, i5_h, i3_h, i1_h, out_h,
            acc, va, xa, vb, xb, v1, x1, sva, sxa, svb, sxb):
        wid = lax.axis_index("s") * 2 + lax.axis_index("c")
        zero = jnp.zeros((16,), jnp.float32)
        vbufs = (va, vb)
        xbufs = (xa, xb)
        svs = (sva, svb)
        sxs = (sxa, sxb)

        sched = []
        for kk, npv, unroll in ((5, 54, 6), (3, 16, 8)):
            for w in range(kk + 1):
                for c0, nc in ((0, 16), (16, 7)):
                    sched.append((kk, w, c0, nc, npv, unroll))

        def task_body(t, _):
            task = wid * 12 + t
            b = task // NG
            r1 = task % NG

            def zbody(i, _):
                for u in range(8):
                    acc[pl.ds(i * 128 + u * 16, 16)] = zero
                return 0
            lax.fori_loop(0, ACC_N // 128, zbody, 0)

            def issue(j, slot):
                kk, w, c0, nc, npv, unroll = sched[j]
                bm_h, i_h = (bm5_h, i5_h) if kk == 5 else (bm3_h, i3_h)
                wdt = 16 * npv
                pad = (kk - 1) // 2
                row = jnp.clip(r1 + pad - kk + w, 0, RES - 1)
                ra = kk - w
                cv = pltpu.async_copy(
                    bm_h.at[b, row, pl.ds(c0, nc), ra, :],
                    vbufs[slot].at[pl.ds(0, nc), pl.ds(0, wdt)], svs[slot])
                cx = pltpu.async_copy(
                    i_h.at[r1, w, pl.ds(c0, nc)],
                    xbufs[slot].at[pl.ds(0, nc), pl.ds(0, wdt)], sxs[slot])
                return cv, cx

            pend = issue(0, 0)
            for j, (kk, w, c0, nc, npv, unroll) in enumerate(sched):
                slot = j % 2
                cv, cx = pend
                cv.wait()
                cx.wait()
                if j + 1 < len(sched):
                    pend = issue(j + 1, 1 - slot)
                vbuf = vbufs[slot]
                xbuf = xbufs[slot]

                def cbody(col, _, vbuf=vbuf, xbuf=xbuf, npv=npv, unroll=unroll):
                    def jbody(jj, _):
                        for u in range(unroll):
                            off = jj * unroll * 16 + u * 16
                            iv = xbuf[col, pl.ds(off, 16)]
                            vv = vbuf[col, pl.ds(off, 16)]
                            plsc.addupdate_scatter(acc, [iv], vv)
                        return 0
                    lax.fori_loop(0, npv // unroll, jbody, 0)
                    return 0
                lax.fori_loop(0, nc, cbody, 0)

            for w in range(2):
                row = jnp.clip(r1 - 1 + w, 0, RES - 1)
                ra = 1 - w
                pltpu.sync_copy(bm1_h.at[b, row, :, ra, :], v1)
                pltpu.sync_copy(i1_h.at[r1, w], x1)

                def c1body(col, _):
                    for u in range(2):
                        iv = x1[col, pl.ds(u * 16, 16)]
                        vv = v1[col, pl.ds(u * 16, 16)]
                        plsc.addupdate_scatter(acc, [iv], vv)
                    return 0
                lax.fori_loop(0, RES, c1body, 0)

            pltpu.sync_copy(acc.at[pl.ds(0, STRIP)], out_h.at[b, r1])
            return 0
        lax.fori_loop(0, 12, task_body, 0)

    i5 = jnp.asarray(_SC_IDX[5])
    i3 = jnp.asarray(_SC_IDX[3])
    i1 = jnp.asarray(_SC_IDX[1])
    return asm(bm5, bm3, bm1, i5, i3, i1)


ROW_BLK = 128


def _mask_body(k_ref, zm_ref, out_ref):
    v = k_ref[...]
    out_ref[...] = jnp.where(zm_ref[...], jnp.zeros((), v.dtype), v)


def _mask(K, zero_map):
    B = K.shape[0]
    spec = pl.BlockSpec((1, ROW_BLK, NDOF), lambda b, r: (b, r, 0))
    return pl.pallas_call(
        _mask_body,
        grid=(B, NDOF // ROW_BLK),
        in_specs=[spec, spec],
        out_specs=spec,
        out_shape=jax.ShapeDtypeStruct((B, NDOF, NDOF), K.dtype),
    )(K, zero_map)


def _im2col(x, k):
    pad = (k - 1) // 2
    B = x.shape[0]
    xp = jnp.pad(x, ((0, 0), (0, 0), (pad, pad), (pad, pad)))
    cols = [xp[:, c, dy:dy + RES, dx:dx + RES]
            for c in range(2) for dy in range(k) for dx in range(k)]
    return jnp.stack(cols, axis=-1).reshape(B, NEL, 2 * k * k)


def kernel(x, zero_map, DBC, f,
           c11_w_1, c11_b_1, c12_w_1, c12_b_1, c13_w_1, c13_b_1, convt_w_1,
           c11_w_3, c11_b_3, c12_w_3, c12_b_3, c13_w_3, c13_b_3, convt_w_3,
           c11_w_5, c11_b_5, c12_w_5, c12_b_5, c13_w_5, c13_b_5, convt_w_5):
    params = {
        1: (c11_w_1, c11_b_1, c12_w_1, c12_b_1, c13_w_1, c13_b_1, convt_w_1),
        3: (c11_w_3, c11_b_3, c12_w_3, c12_b_3, c13_w_3, c13_b_3, convt_w_3),
        5: (c11_w_5, c11_b_5, c12_w_5, c12_b_5, c13_w_5, c13_b_5, convt_w_5),
    }
    bms = {}
    for k in KERNELS:
        w1, b1, w2, b2, w3, b3, cw = params[k]
        wi = w1.shape[0]
        out_size = 2 * (k + 1) ** 2
        W1 = w1.reshape(wi, 2 * k * k).T
        W2 = w2[:, :, 0, 0].T
        W3 = w3[:, :, 0, 0].T
        Wm = cw[:, 0, 0, :]
        Wm2 = jnp.zeros((2 * wi, out_size * out_size), Wm.dtype)
        Wm2 = Wm2.at[:, _tril_cols(out_size)].set(Wm)
        patches = _im2col(x, k)
        bm = _tower(k, patches, W1, b1.reshape(1, -1), W2, b2.reshape(1, -1),
                    W3, b3.reshape(1, -1), Wm2)
        s = k + 1
        L = 2 * s * s
        bms[k] = bm.reshape(x.shape[0], RES, RES, s, 2 * s * L)
    raw = _sc_assemble(bms[5], bms[3], bms[1])
    K = raw.reshape(x.shape[0], NDOF, NDOF)
    return _mask(K, zero_map)


# speedup vs baseline: 1.0604x; 1.0557x over previous
import functools
import numpy as np
import jax
import jax.numpy as jnp
from jax import lax
from jax.experimental import pallas as pl
from jax.experimental.pallas import tpu as pltpu
from jax.experimental.pallas import tpu_sc as plsc

RES = 23
KERNELS = [1, 3, 5]
WBASE = 16
NG = RES + 1
NDOF = 2 * NG * NG
NEL = RES * RES
STRIP = 48 * NDOF
DUMP = STRIP
ACC_N = STRIP + 128


def _build_sc_idx(k):
    pad = (k - 1) // 2
    s = k + 1
    L = 2 * s * s
    sh = (NG, s, RES, s, 2, s, s, 2)
    R1, W, COL, CA, PA, RB, CB, PB = np.ogrid[0:NG, 0:s, 0:RES, 0:s, 0:2, 0:s, 0:s, 0:2]
    row = R1 + pad - k + W
    c1 = COL + CA - pad
    r2 = row + RB - pad
    c2 = COL + CB - pad
    ok = ((row >= 0) & (row <= RES - 1) & (c1 >= 0) & (c1 < NG)
          & (r2 >= 0) & (r2 < NG) & (c2 >= 0) & (c2 < NG))
    val = (2 * c1 + PA) * NDOF + 48 * r2 + 2 * c2 + PB
    flat = ((COL * 2 * s + CA * 2 + PA) * L + (RB * s + CB) * 2 + PB) % 16
    idx = np.where(ok, val, DUMP + flat)
    idx = np.broadcast_to(idx, sh)
    return np.ascontiguousarray(idx.reshape(NG, s, RES, 2 * s * L).astype(np.int32))


_SC_IDX = {k: _build_sc_idx(k) for k in KERNELS}


def _tril_cols(out_size):
    tri = np.tril_indices(out_size)
    return tri[0] * out_size + tri[1]


def _tower_body(Lk, p_ref, w1_ref, b1_ref, w2_ref, b2_ref, w3_ref, b3_ref,
                wm_ref, out_ref):
    P = p_ref[0]
    H1 = jnp.maximum(jnp.dot(P, w1_ref[...], preferred_element_type=jnp.float32)
                     + b1_ref[...], 0.0)
    H2 = jnp.maximum(jnp.dot(H1, w2_ref[...], preferred_element_type=jnp.float32)
                     + b2_ref[...], 0.0)
    H3 = jnp.maximum(jnp.dot(H2, w3_ref[...], preferred_element_type=jnp.float32)
                     + b3_ref[...], 0.0)
    Bm = jnp.dot(H3, wm_ref[...], preferred_element_type=jnp.float32)
    ii = lax.broadcasted_iota(jnp.int32, (1, Lk * Lk), 1)
    dmask = (ii // Lk) == (ii % Lk)
    Bm = jnp.where(dmask, jnp.maximum(Bm, 0.0), Bm)
    Bm3 = Bm.reshape(NEL, Lk, Lk)
    bm = lax.dot_general(Bm3, Bm3, (((2,), (2,)), ((0,), (0,))),
                         preferred_element_type=jnp.float32)
    out_ref[0] = bm


def _tower(k, patches, w1, b1, w2, b2, w3, b3, wm2):
    B = patches.shape[0]
    Ck = patches.shape[2]
    Lk = 2 * (k + 1) ** 2
    wi = w1.shape[1]
    full = lambda shp: pl.BlockSpec(shp, lambda b: (0,) * len(shp))
    return pl.pallas_call(
        functools.partial(_tower_body, Lk),
        grid=(B,),
        in_specs=[
            pl.BlockSpec((1, NEL, Ck), lambda b: (b, 0, 0)),
            full((Ck, wi)), full((1, wi)),
            full((wi, 2 * wi)), full((1, 2 * wi)),
            full((2 * wi, 2 * wi)), full((1, 2 * wi)),
            full((2 * wi, Lk * Lk)),
        ],
        out_specs=pl.BlockSpec((1, NEL, Lk, Lk), lambda b: (b, 0, 0, 0)),
        out_shape=jax.ShapeDtypeStruct((B, NEL, Lk, Lk), jnp.float32),
        compiler_params=pltpu.CompilerParams(
            vmem_limit_bytes=100 * 1024 * 1024),
    )(patches, w1, b1, w2, b2, w3, b3, wm2)


def _sc_assemble(bm5, bm3, bm1):
    mesh = plsc.VectorSubcoreMesh(core_axis_name="c", subcore_axis_name="s")
    B = bm5.shape[0]

    @functools.partial(
        pl.kernel,
        out_type=jax.ShapeDtypeStruct((B, NG, STRIP), jnp.float32),
        mesh=mesh,
        compiler_params=pltpu.CompilerParams(needs_layout_passes=False),
        scratch_types=[
            pltpu.VMEM((ACC_N,), jnp.float32),
            pltpu.VMEM((16, 864), jnp.float32), pltpu.VMEM((16, 864), jnp.int32),
            pltpu.VMEM((16, 864), jnp.float32), pltpu.VMEM((16, 864), jnp.int32),
            pltpu.VMEM((RES, 32), jnp.float32), pltpu.VMEM((RES, 32), jnp.int32),
            pltpu.SemaphoreType.DMA, pltpu.SemaphoreType.DMA,
            pltpu.SemaphoreType.DMA, pltpu.SemaphoreType.DMA,
        ],
    )
    def asm(bm5_h, bm3_h, bm1_h, i5_h, i3_h, i1_h, out_h,
            acc, va, xa, vb, xb, v1, x1, sva, sxa, svb, sxb):
        wid = lax.axis_index("s") * 2 + lax.axis_index("c")
        zero = jnp.zeros((16,), jnp.float32)
        vbufs = (va, vb)
        xbufs = (xa, xb)
        svs = (sva, svb)
        sxs = (sxa, sxb)

        sched = []
        for kk, npv, unroll in ((5, 54, 6), (3, 16, 8)):
            for w in range(kk + 1):
                for c0, nc in ((0, 16), (16, 7)):
                    sched.append((kk, w, c0, nc, npv, unroll))

        def task_body(t, _):
            task = wid * 12 + t
            b = task // NG
            r1 = task % NG

            def zbody(i, _):
                for u in range(8):
                    acc[pl.ds(i * 128 + u * 16, 16)] = zero
                return 0
            lax.fori_loop(0, ACC_N // 128, zbody, 0)

            def issue(j, slot):
                kk, w, c0, nc, npv, unroll = sched[j]
                bm_h, i_h = (bm5_h, i5_h) if kk == 5 else (bm3_h, i3_h)
                wdt = 16 * npv
                pad = (kk - 1) // 2
                row = jnp.clip(r1 + pad - kk + w, 0, RES - 1)
                ra = kk - w
                cv = pltpu.async_copy(
                    bm_h.at[b, row, pl.ds(c0, nc), ra, :],
                    vbufs[slot].at[pl.ds(0, nc), pl.ds(0, wdt)], svs[slot])
                cx = pltpu.async_copy(
                    i_h.at[r1, w, pl.ds(c0, nc)],
                    xbufs[slot].at[pl.ds(0, nc), pl.ds(0, wdt)], sxs[slot])
                return cv, cx

            pend = issue(0, 0)
            for j, (kk, w, c0, nc, npv, unroll) in enumerate(sched):
                slot = j % 2
                cv, cx = pend
                cv.wait()
                cx.wait()
                if j + 1 < len(sched):
                    pend = issue(j + 1, 1 - slot)
                vbuf = vbufs[slot]
                xbuf = xbufs[slot]

                def cbody(col, _, vbuf=vbuf, xbuf=xbuf, npv=npv, unroll=unroll):
                    def jbody(jj, _):
                        for u in range(unroll):
                            off = jj * unroll * 16 + u * 16
                            iv = xbuf[col, pl.ds(off, 16)]
                            vv = vbuf[col, pl.ds(off, 16)]
                            plsc.addupdate_scatter(acc, [iv], vv)
                        return 0
                    lax.fori_loop(0, npv // unroll, jbody, 0)
                    return 0
                lax.fori_loop(0, nc, cbody, 0)

            for w in range(2):
                row = jnp.clip(r1 - 1 + w, 0, RES - 1)
                ra = 1 - w
                pltpu.sync_copy(bm1_h.at[b, row, :, ra, :], v1)
                pltpu.sync_copy(i1_h.at[r1, w], x1)

                def c1body(col, _):
                    for u in range(2):
                        iv = x1[col, pl.ds(u * 16, 16)]
                        vv = v1[col, pl.ds(u * 16, 16)]
                        plsc.addupdate_scatter(acc, [iv], vv)
                    return 0
                lax.fori_loop(0, RES, c1body, 0)

            pltpu.sync_copy(acc.at[pl.ds(0, STRIP)], out_h.at[b, r1])
            return 0
        lax.fori_loop(0, 12, task_body, 0)

    i5 = jnp.asarray(_SC_IDX[5])
    i3 = jnp.asarray(_SC_IDX[3])
    i1 = jnp.asarray(_SC_IDX[1])
    return asm(bm5, bm3, bm1, i5, i3, i1)


ROW_BLK = 128


def _mask_body(k_ref, zm_ref, out_ref):
    v = k_ref[...]
    out_ref[...] = jnp.where(zm_ref[...], jnp.zeros((), v.dtype), v)


def _mask(K, zero_map):
    B = K.shape[0]
    spec = pl.BlockSpec((1, ROW_BLK, NDOF), lambda b, r: (b, r, 0))
    return pl.pallas_call(
        _mask_body,
        grid=(B, NDOF // ROW_BLK),
        in_specs=[spec, spec],
        out_specs=spec,
        out_shape=jax.ShapeDtypeStruct((B, NDOF, NDOF), K.dtype),
    )(K, zero_map)


def _im2col(x, k):
    pad = (k - 1) // 2
    B = x.shape[0]
    xp = jnp.pad(x, ((0, 0), (0, 0), (pad, pad), (pad, pad)))
    cols = [xp[:, c, dy:dy + RES, dx:dx + RES]
            for c in range(2) for dy in range(k) for dx in range(k)]
    return jnp.stack(cols, axis=-1).reshape(B, NEL, 2 * k * k)


def kernel(x, zero_map, DBC, f,
           c11_w_1, c11_b_1, c12_w_1, c12_b_1, c13_w_1, c13_b_1, convt_w_1,
           c11_w_3, c11_b_3, c12_w_3, c12_b_3, c13_w_3, c13_b_3, convt_w_3,
           c11_w_5, c11_b_5, c12_w_5, c12_b_5, c13_w_5, c13_b_5, convt_w_5):
    params = {
        1: (c11_w_1, c11_b_1, c12_w_1, c12_b_1, c13_w_1, c13_b_1, convt_w_1),
        3: (c11_w_3, c11_b_3, c12_w_3, c12_b_3, c13_w_3, c13_b_3, convt_w_3),
        5: (c11_w_5, c11_b_5, c12_w_5, c12_b_5, c13_w_5, c13_b_5, convt_w_5),
    }
    bms = {}
    for k in KERNELS:
        w1, b1, w2, b2, w3, b3, cw = params[k]
        wi = w1.shape[0]
        out_size = 2 * (k + 1) ** 2
        W1 = w1.reshape(wi, 2 * k * k).T
        W2 = w2[:, :, 0, 0].T
        W3 = w3[:, :, 0, 0].T
        Wm = cw[:, 0, 0, :]
        Wm2 = jnp.zeros((2 * wi, out_size * out_size), Wm.dtype)
        Wm2 = Wm2.at[:, _tril_cols(out_size)].set(Wm)
        patches = _im2col(x, k)
        bm = _tower(k, patches, W1, b1.reshape(1, -1), W2, b2.reshape(1, -1),
                    W3, b3.reshape(1, -1), Wm2)
        s = k + 1
        L = 2 * s * s
        bms[k] = bm.reshape(x.shape[0], RES, RES, s, 2 * s * L)
    raw = _sc_assemble(bms[5], bms[3], bms[1])
    K = raw.reshape(x.shape[0], NDOF, NDOF)
    return _mask(K, zero_map)
